# baseline (speedup 1.0000x reference)
import functools

import jax
import jax.numpy as jnp
from jax import lax
from jax.experimental import pallas as pl
from jax.experimental.pallas import tpu as pltpu
from jax.experimental.pallas import tpu_sc as plsc


def _sc_gather(time_emb, timestep):
    B = timestep.shape[0]
    C = time_emb.shape[1]
    rows_per_worker = 8
    n_workers = B // rows_per_worker
    mesh = plsc.VectorSubcoreMesh(core_axis_name="c", subcore_axis_name="s")
    info = plsc.get_sparse_core_info()
    nc = info.num_cores

    @functools.partial(
        pl.kernel,
        mesh=mesh,
        out_type=jax.ShapeDtypeStruct((B, C), jnp.float32),
        scratch_types=[
            pltpu.VMEM((rows_per_worker,), jnp.int32),
            pltpu.VMEM((rows_per_worker, C), jnp.float32),
            pltpu.SemaphoreType.DMA,
        ],
    )
    def gather(table_hbm, idx_hbm, out_hbm, idx_v, rows_v, sem):
        wid = lax.axis_index("s") * nc + lax.axis_index("c")

        @pl.when(wid < n_workers)
        def _():
            base = wid * rows_per_worker
            pltpu.sync_copy(idx_hbm.at[pl.ds(base, rows_per_worker)], idx_v)
            pltpu.async_copy(table_hbm.at[idx_v], rows_v, sem).wait()
            pltpu.sync_copy(rows_v, out_hbm.at[pl.ds(base, rows_per_worker)])

    return gather(time_emb, timestep)


_CHUNK_ROWS = 256
_DEPTH = 8


def _make_stream_body(n_chunks, chunks_per_b, hw):
    cr = _CHUNK_ROWS

    def body(time_t_ref, x_hbm, out_hbm, inbuf, outbuf, insem, outsem):
        def in_copy(chunk, slot):
            b = chunk // chunks_per_b
            c0 = lax.rem(chunk, chunks_per_b) * cr
            return pltpu.make_async_copy(
                x_hbm.at[b, pl.ds(c0, cr), :], inbuf.at[slot], insem.at[slot]
            )

        def out_copy(chunk, slot):
            b = chunk // chunks_per_b
            c0 = lax.rem(chunk, chunks_per_b) * cr
            return pltpu.make_async_copy(
                outbuf.at[slot], out_hbm.at[b, pl.ds(c0, cr), :], outsem.at[slot]
            )

        def start_in(chunk, slot):
            in_copy(chunk, slot).start(priority=slot % 2)

        def wait_in(chunk, slot):
            in_copy(chunk, slot).wait()

        def start_out(chunk, slot):
            out_copy(chunk, slot).start(priority=slot % 2)

        def wait_out(chunk, slot):
            out_copy(chunk, slot).wait()

        for j in range(_DEPTH):
            start_in(j, j)

        n_groups = n_chunks // _DEPTH

        def step(g, _):
            for j in range(_DEPTH):
                i = g * _DEPTH + j
                b = i // chunks_per_b
                c0 = lax.rem(i, chunks_per_b) * cr
                wait_in(i, j)

                @pl.when(g > 0)
                def _():
                    wait_out(i - _DEPTH, j)

                tcol = time_t_ref[pl.ds(c0, cr), :]
                lane = lax.broadcasted_iota(jnp.int32, tcol.shape, 1)
                vals = jnp.sum(
                    jnp.where(lane == b, tcol, 0.0), axis=1, keepdims=True
                )
                outbuf[j] = inbuf[j] + vals
                start_out(i, j)

                @pl.when(i + _DEPTH < n_chunks)
                def _():
                    start_in(i + _DEPTH, j)

            return 0

        lax.fori_loop(0, n_groups, step, 0)
        for j in range(_DEPTH):
            wait_out(n_chunks - _DEPTH + j, j)

    return body


def kernel(x, timestep, time_emb):
    B, C, H, W = x.shape
    HW = H * W
    time = _sc_gather(time_emb, timestep.astype(jnp.int32))
    time_t = time.T
    x3 = x.reshape(B, C, HW)
    chunks_per_b = C // _CHUNK_ROWS
    n_chunks = (B * C) // _CHUNK_ROWS
    out = pl.pallas_call(
        _make_stream_body(n_chunks, chunks_per_b, HW),
        in_specs=[
            pl.BlockSpec(memory_space=pltpu.VMEM),
            pl.BlockSpec(memory_space=pltpu.HBM),
        ],
        out_specs=pl.BlockSpec(memory_space=pltpu.HBM),
        out_shape=jax.ShapeDtypeStruct((B, C, HW), x.dtype),
        scratch_shapes=[
            pltpu.VMEM((_DEPTH, _CHUNK_ROWS, HW), jnp.float32),
            pltpu.VMEM((_DEPTH, _CHUNK_ROWS, HW), jnp.float32),
            pltpu.SemaphoreType.DMA((_DEPTH,)),
            pltpu.SemaphoreType.DMA((_DEPTH,)),
        ],
    )(time_t, x3)
    return out.reshape(B, C, H, W)


# speedup vs baseline: 1.0016x; 1.0016x over previous
import functools

import jax
import jax.numpy as jnp
from jax import lax
from jax.experimental import pallas as pl
from jax.experimental.pallas import tpu as pltpu
from jax.experimental.pallas import tpu_sc as plsc


def _sc_gather(time_emb, timestep):
    B = timestep.shape[0]
    C = time_emb.shape[1]
    rows_per_worker = 8
    n_workers = B // rows_per_worker
    mesh = plsc.VectorSubcoreMesh(core_axis_name="c", subcore_axis_name="s")
    info = plsc.get_sparse_core_info()
    nc = info.num_cores

    @functools.partial(
        pl.kernel,
        mesh=mesh,
        out_type=jax.ShapeDtypeStruct((B, C), jnp.float32),
        scratch_types=[
            pltpu.VMEM((rows_per_worker,), jnp.int32),
            pltpu.VMEM((rows_per_worker, C), jnp.float32),
            pltpu.SemaphoreType.DMA,
        ],
    )
    def gather(table_hbm, idx_hbm, out_hbm, idx_v, rows_v, sem):
        wid = lax.axis_index("s") * nc + lax.axis_index("c")

        @pl.when(wid < n_workers)
        def _():
            base = wid * rows_per_worker
            pltpu.sync_copy(idx_hbm.at[pl.ds(base, rows_per_worker)], idx_v)
            pltpu.async_copy(table_hbm.at[idx_v], rows_v, sem).wait()
            pltpu.sync_copy(rows_v, out_hbm.at[pl.ds(base, rows_per_worker)])

    return gather(time_emb, timestep)


_CHUNK_ROWS = 768
_DEPTH = 4


def _make_stream_body(n_chunks, chunks_per_b, hw):
    cr = _CHUNK_ROWS

    def body(time_t_ref, x_hbm, out_hbm, inbuf, outbuf, insem, outsem):
        def perm(i):
            return lax.rem(i, _DEPTH) * (n_chunks // _DEPTH) + i // _DEPTH

        def in_copy(chunk, slot):
            b = chunk // chunks_per_b
            c0 = lax.rem(chunk, chunks_per_b) * cr
            return pltpu.make_async_copy(
                x_hbm.at[b, pl.ds(c0, cr), :], inbuf.at[slot], insem.at[slot]
            )

        def out_copy(chunk, slot):
            b = chunk // chunks_per_b
            c0 = lax.rem(chunk, chunks_per_b) * cr
            return pltpu.make_async_copy(
                outbuf.at[slot], out_hbm.at[b, pl.ds(c0, cr), :], outsem.at[slot]
            )

        def start_in(chunk, slot):
            in_copy(chunk, slot).start(priority=slot % 2)

        def wait_in(chunk, slot):
            in_copy(chunk, slot).wait()

        def start_out(chunk, slot):
            out_copy(chunk, slot).start(priority=slot % 2)

        def wait_out(chunk, slot):
            out_copy(chunk, slot).wait()

        for j in range(_DEPTH):
            start_in(perm(j), j)

        n_groups = n_chunks // _DEPTH

        def step(g, _):
            for j in range(_DEPTH):
                i = g * _DEPTH + j
                chunk = perm(i)
                b = chunk // chunks_per_b
                c0 = lax.rem(chunk, chunks_per_b) * cr
                wait_in(chunk, j)

                @pl.when(g > 0)
                def _():
                    wait_out(perm(i - _DEPTH), j)

                tcol = time_t_ref[pl.ds(c0, cr), :]
                lane = lax.broadcasted_iota(jnp.int32, tcol.shape, 1)
                vals = jnp.sum(
                    jnp.where(lane == b, tcol, 0.0), axis=1, keepdims=True
                )
                outbuf[j] = inbuf[j] + vals
                start_out(chunk, j)

                @pl.when(i + _DEPTH < n_chunks)
                def _():
                    start_in(perm(i + _DEPTH), j)

            return 0

        lax.fori_loop(0, n_groups, step, 0)
        for j in range(_DEPTH):
            wait_out(perm(n_chunks - _DEPTH + j), j)

    return body


def kernel(x, timestep, time_emb):
    B, C, H, W = x.shape
    HW = H * W
    time = _sc_gather(time_emb, timestep.astype(jnp.int32))
    time_t = time.T
    x3 = x.reshape(B, C, HW)
    chunks_per_b = C // _CHUNK_ROWS
    n_chunks = (B * C) // _CHUNK_ROWS
    out = pl.pallas_call(
        _make_stream_body(n_chunks, chunks_per_b, HW),
        in_specs=[
            pl.BlockSpec(memory_space=pltpu.VMEM),
            pl.BlockSpec(memory_space=pltpu.HBM),
        ],
        out_specs=pl.BlockSpec(memory_space=pltpu.HBM),
        out_shape=jax.ShapeDtypeStruct((B, C, HW), x.dtype),
        scratch_shapes=[
            pltpu.VMEM((_DEPTH, _CHUNK_ROWS, HW), jnp.float32),
            pltpu.VMEM((_DEPTH, _CHUNK_ROWS, HW), jnp.float32),
            pltpu.SemaphoreType.DMA((_DEPTH,)),
            pltpu.SemaphoreType.DMA((_DEPTH,)),
        ],
    )(time_t, x3)
    return out.reshape(B, C, H, W)
